# Initial kernel scaffold; baseline (speedup 1.0000x reference)
import functools

import jax
import jax.numpy as jnp
from jax import lax
from jax.experimental import pallas as pl
from jax.experimental.pallas import tpu as pltpu
from jax.experimental.pallas import tpu_sc as plsc

_LANES = 128
_CHUNK_IR = 8


@functools.lru_cache(maxsize=None)
def _make_gather(n_table_rows, dim, n_idx_rows):
    info = plsc.get_sparse_core_info()
    nw = info.num_cores * info.num_subcores
    rows_per_w = n_idx_rows // nw
    n_chunks = rows_per_w // _CHUNK_IR
    chunk_labels = _CHUNK_IR * _LANES
    mesh = plsc.VectorSubcoreMesh(core_axis_name="c", subcore_axis_name="s")

    @functools.partial(
        pl.kernel,
        mesh=mesh,
        out_type=jax.ShapeDtypeStruct((n_idx_rows * _LANES, dim), jnp.float32),
        scratch_types=[
            pltpu.VMEM((rows_per_w, _LANES), jnp.int32),
            pltpu.VMEM((chunk_labels, dim), jnp.float32),
            pltpu.SemaphoreType.DMA,
        ],
    )
    def gather_kernel(table_hbm, idx_hbm, out_hbm, idx_v, rows_v, sem):
        num_cores = info.num_cores
        wid = lax.axis_index("s") * num_cores + lax.axis_index("c")
        ir0 = wid * rows_per_w
        pltpu.sync_copy(idx_hbm.at[pl.ds(ir0, rows_per_w)], idx_v)

        def body(g, carry):
            copies = []
            for j in range(_CHUNK_IR):
                copies.append(pltpu.async_copy(
                    table_hbm.at[idx_v.at[g * _CHUNK_IR + j]],
                    rows_v.at[pl.ds(j * _LANES, _LANES)],
                    sem,
                ))
            for c in copies:
                c.wait()
            row0 = (ir0 + g * _CHUNK_IR) * _LANES
            pltpu.sync_copy(rows_v, out_hbm.at[pl.ds(row0, chunk_labels)])
            return carry

        lax.fori_loop(0, n_chunks, body, 0)

    return gather_kernel


def kernel(label, table):
    n_labels = label.size
    n_idx_rows = n_labels // _LANES
    flat = label.reshape(n_idx_rows, _LANES).astype(jnp.int32)
    out = _make_gather(table.shape[0], table.shape[1], n_idx_rows)(table, flat)
    return out.reshape(label.shape + (table.shape[1],))


# speedup vs baseline: 1.1023x; 1.1023x over previous
import functools

import jax
import jax.numpy as jnp
from jax import lax
from jax.experimental import pallas as pl
from jax.experimental.pallas import tpu as pltpu
from jax.experimental.pallas import tpu_sc as plsc

_LANES = 128
_CHUNK_IR = 8


@functools.lru_cache(maxsize=None)
def _make_gather(n_table_rows, dim, n_idx_rows):
    info = plsc.get_sparse_core_info()
    nw = info.num_cores * info.num_subcores
    rows_per_w = n_idx_rows // nw
    n_chunks = rows_per_w // _CHUNK_IR
    chunk_labels = _CHUNK_IR * _LANES
    mesh = plsc.VectorSubcoreMesh(core_axis_name="c", subcore_axis_name="s")

    @functools.partial(
        pl.kernel,
        mesh=mesh,
        compiler_params=pltpu.CompilerParams(use_tc_tiling_on_sc=False),
        out_type=jax.ShapeDtypeStruct((n_idx_rows * _LANES, dim), jnp.float32),
        scratch_types=[
            pltpu.VMEM((rows_per_w, _LANES), jnp.int32),
            pltpu.VMEM((chunk_labels, dim), jnp.float32),
            pltpu.SemaphoreType.DMA,
        ],
    )
    def gather_kernel(table_hbm, idx_hbm, out_hbm, idx_v, rows_v, sem):
        num_cores = info.num_cores
        wid = lax.axis_index("s") * num_cores + lax.axis_index("c")
        ir0 = wid * rows_per_w
        pltpu.sync_copy(idx_hbm.at[pl.ds(ir0, rows_per_w)], idx_v)

        def body(g, carry):
            copies = []
            for j in range(_CHUNK_IR):
                copies.append(pltpu.async_copy(
                    table_hbm.at[idx_v.at[g * _CHUNK_IR + j]],
                    rows_v.at[pl.ds(j * _LANES, _LANES)],
                    sem,
                ))
            for c in copies:
                c.wait()
            row0 = (ir0 + g * _CHUNK_IR) * _LANES
            pltpu.sync_copy(rows_v, out_hbm.at[pl.ds(row0, chunk_labels)])
            return carry

        lax.fori_loop(0, n_chunks, body, 0)

    return gather_kernel


def kernel(label, table):
    n_labels = label.size
    n_idx_rows = n_labels // _LANES
    flat = label.reshape(n_idx_rows, _LANES).astype(jnp.int32)
    out = _make_gather(table.shape[0], table.shape[1], n_idx_rows)(table, flat)
    return out.reshape(label.shape + (table.shape[1],))


# speedup vs baseline: 1.1130x; 1.0097x over previous
import functools

import jax
import jax.numpy as jnp
from jax import lax
from jax.experimental import pallas as pl
from jax.experimental.pallas import tpu as pltpu
from jax.experimental.pallas import tpu_sc as plsc

_LANES = 128
_CHUNK_IR = 4
_NBUF = 5


@functools.lru_cache(maxsize=None)
def _make_gather(n_table_rows, dim, n_idx_rows):
    info = plsc.get_sparse_core_info()
    nw = info.num_cores * info.num_subcores
    rows_per_w = n_idx_rows // nw
    n_chunks = rows_per_w // _CHUNK_IR
    chunk_rows = _CHUNK_IR * _LANES
    assert rows_per_w % _CHUNK_IR == 0 and n_chunks % _NBUF == 0
    mesh = plsc.VectorSubcoreMesh(core_axis_name="c", subcore_axis_name="s")

    @functools.partial(
        pl.kernel,
        mesh=mesh,
        compiler_params=pltpu.CompilerParams(use_tc_tiling_on_sc=False),
        out_type=jax.ShapeDtypeStruct((n_idx_rows * _LANES, dim), jnp.float32),
        scratch_types=[
            pltpu.VMEM((rows_per_w, _LANES), jnp.int32),
            *[pltpu.VMEM((chunk_rows, dim), jnp.float32) for _ in range(_NBUF)],
            *[pltpu.SemaphoreType.DMA for _ in range(_NBUF)],
            *[pltpu.SemaphoreType.DMA for _ in range(_NBUF)],
        ],
    )
    def gather_kernel(table_hbm, idx_hbm, out_hbm, idx_v, *rest):
        bufs = rest[:_NBUF]
        gsem = rest[_NBUF:2 * _NBUF]
        wsem = rest[2 * _NBUF:]
        wid = lax.axis_index("s") * info.num_cores + lax.axis_index("c")
        ir0 = wid * rows_per_w
        row0 = ir0 * _LANES
        pltpu.sync_copy(idx_hbm.at[pl.ds(ir0, rows_per_w)], idx_v)

        def fire_gather(b, c):
            for j in range(_CHUNK_IR):
                pltpu.async_copy(
                    table_hbm.at[idx_v.at[c * _CHUNK_IR + j]],
                    bufs[b].at[pl.ds(j * _LANES, _LANES)],
                    gsem[b],
                )

        def wait_gather(b):
            pltpu.make_async_copy(
                table_hbm.at[pl.ds(0, chunk_rows)], bufs[b], gsem[b]).wait()

        def fire_write(b, c):
            pltpu.async_copy(
                bufs[b], out_hbm.at[pl.ds(row0 + c * chunk_rows, chunk_rows)],
                wsem[b])

        def wait_write(b):
            pltpu.make_async_copy(
                bufs[b], out_hbm.at[pl.ds(0, chunk_rows)], wsem[b]).wait()

        for b in range(_NBUF):
            fire_gather(b, b)

        def body(i, carry):
            g = i * _NBUF
            for b in range(_NBUF):
                c = g + b
                bp = (b - 1) % _NBUF
                wait_gather(b)
                fire_write(b, c)
                if b == 0:
                    @pl.when(g > 0)
                    def _():
                        wait_write(bp)
                        fire_gather(bp, c - 1 + _NBUF)
                else:
                    wait_write(bp)
                    cn = c - 1 + _NBUF
                    @pl.when(cn < n_chunks)
                    def _():
                        fire_gather(bp, cn)
            return carry

        lax.fori_loop(0, n_chunks // _NBUF, body, 0)
        wait_write((n_chunks - 1) % _NBUF)

    return gather_kernel


def kernel(label, table):
    n_labels = label.size
    n_idx_rows = n_labels // _LANES
    flat = label.reshape(n_idx_rows, _LANES).astype(jnp.int32)
    out = _make_gather(table.shape[0], table.shape[1], n_idx_rows)(table, flat)
    return out.reshape(label.shape + (table.shape[1],))


# speedup vs baseline: 1.8057x; 1.6224x over previous
import functools

import jax
import jax.numpy as jnp
from jax import lax
from jax.experimental import pallas as pl
from jax.experimental.pallas import tpu as pltpu
from jax.experimental.pallas import tpu_sc as plsc

_CHUNK_LR = 8
_NBUF = 4


@functools.lru_cache(maxsize=None)
def _make_gather(n_table_rows, dim, n_label_rows, n_cols):
    info = plsc.get_sparse_core_info()
    nw = info.num_cores * info.num_subcores
    rows_per_w = n_label_rows // nw
    n_chunks = rows_per_w // _CHUNK_LR
    assert rows_per_w % _CHUNK_LR == 0 and n_chunks % _NBUF == 0
    mesh = plsc.VectorSubcoreMesh(core_axis_name="c", subcore_axis_name="s")

    @functools.partial(
        pl.kernel,
        mesh=mesh,
        compiler_params=pltpu.CompilerParams(use_tc_tiling_on_sc=False),
        out_type=jax.ShapeDtypeStruct((n_label_rows, n_cols, dim), jnp.float32),
        scratch_types=[
            pltpu.VMEM((rows_per_w, n_cols), jnp.int32),
            *[pltpu.VMEM((_CHUNK_LR, n_cols, dim), jnp.float32)
              for _ in range(_NBUF)],
            *[pltpu.SemaphoreType.DMA for _ in range(_NBUF)],
            *[pltpu.SemaphoreType.DMA for _ in range(_NBUF)],
        ],
    )
    def gather_kernel(table_hbm, idx_hbm, out_hbm, idx_v, *rest):
        bufs = rest[:_NBUF]
        gsem = rest[_NBUF:2 * _NBUF]
        wsem = rest[2 * _NBUF:]
        wid = lax.axis_index("s") * info.num_cores + lax.axis_index("c")
        lr0 = wid * rows_per_w
        pltpu.sync_copy(idx_hbm.at[pl.ds(lr0, rows_per_w)], idx_v)

        def fire_gather(b, c):
            for j in range(_CHUNK_LR):
                pltpu.async_copy(
                    table_hbm.at[idx_v.at[c * _CHUNK_LR + j]],
                    bufs[b].at[j],
                    gsem[b],
                )

        def wait_gather(b):
            pltpu.make_async_copy(
                out_hbm.at[pl.ds(0, _CHUNK_LR)], bufs[b], gsem[b]).wait()

        def fire_write(b, c):
            pltpu.async_copy(
                bufs[b], out_hbm.at[pl.ds(lr0 + c * _CHUNK_LR, _CHUNK_LR)],
                wsem[b])

        def wait_write(b):
            pltpu.make_async_copy(
                bufs[b], out_hbm.at[pl.ds(0, _CHUNK_LR)], wsem[b]).wait()

        for b in range(_NBUF):
            fire_gather(b, b)

        def body(i, carry):
            g = i * _NBUF
            for b in range(_NBUF):
                c = g + b
                bp = (b - 1) % _NBUF
                wait_gather(b)
                fire_write(b, c)
                if b == 0:
                    @pl.when(g > 0)
                    def _():
                        wait_write(bp)
                        fire_gather(bp, c - 1 + _NBUF)
                else:
                    wait_write(bp)
                    cn = c - 1 + _NBUF
                    @pl.when(cn < n_chunks)
                    def _():
                        fire_gather(bp, cn)
            return carry

        lax.fori_loop(0, n_chunks // _NBUF, body, 0)
        wait_write((n_chunks - 1) % _NBUF)

    return gather_kernel


def kernel(label, table):
    n_label_rows, n_cols = label.shape
    return _make_gather(table.shape[0], table.shape[1],
                        n_label_rows, n_cols)(table, label)
